# baseline (speedup 1.0000x reference)
import functools

import jax
import jax.numpy as jnp
from jax import lax
from jax.experimental import pallas as pl
from jax.experimental.pallas import tpu as pltpu
from jax.experimental.pallas import tpu_sc as plsc

N_NODES = 10000
E_EDGES = 320000
D = 128
DE = 16
G_SEG = 128
BN_EPS = 1e-05

NC, NS = 2, 16
NW = NC * NS
N_PAD = 10240
ROWS_PS = N_PAD // NS
EPT = E_EDGES // NW
CH = 80
NCHUNK = EPT // CH

NB = 8
RB = N_PAD // NB
EB = 3200


def _edge_body(ea_ref, W_ref, b_ref, e_ref):
    a = ea_ref[...]
    e_ref[...] = jnp.dot(a, W_ref[...], preferred_element_type=jnp.float32) + b_ref[...]


def _edge_transform(edge_attr, We, be):
    grid = E_EDGES // EB
    return pl.pallas_call(
        _edge_body,
        grid=(grid,),
        in_specs=[
            pl.BlockSpec((EB, DE), lambda i: (i, 0)),
            pl.BlockSpec((DE, D), lambda i: (0, 0)),
            pl.BlockSpec((1, D), lambda i: (0, 0)),
        ],
        out_specs=pl.BlockSpec((EB, D), lambda i: (i, 0)),
        out_shape=jax.ShapeDtypeStruct((E_EDGES, D), jnp.float32),
    )(edge_attr, We, be.reshape(1, D))


def _sc_msg_body(x_hbm, src_hbm, dst_hbm, e_hbm, out_hbm,
                 sidx_all, didx_all, sidx_a, didx_a, sidx_b, didx_b,
                 rows_a, rows_b, aggr_sh,
                 sem_l, sem_ea, sem_eb, sem_ga, sem_gb, sem_sa, sem_sb):
    c = lax.axis_index("c")
    s = lax.axis_index("s")
    w = c * NS + s
    row0 = s * ROWS_PS
    ebase0 = w * EPT

    pltpu.sync_copy(src_hbm.at[w], sidx_all)
    pltpu.sync_copy(dst_hbm.at[w], didx_all)

    def zstore(r, carry):
        for k in range(D // 16):
            rows_a[r, pl.ds(k * 16, 16)] = jnp.zeros((16,), jnp.float32)
        return carry

    lax.fori_loop(0, CH, zstore, 0)
    for t in range(ROWS_PS // CH):
        pltpu.sync_copy(rows_a, aggr_sh.at[pl.ds(row0 + t * CH, CH)])

    def fill(dst_ref, src_ref, j):
        for k in range(CH // 16):
            dst_ref[pl.ds(k * 16, 16)] = src_ref[pl.ds(j * CH + k * 16, 16)]

    def e_load(buf, sem, j):
        eb = ebase0 + lax.rem(j, NCHUNK) * CH
        return pltpu.async_copy(e_hbm.at[pl.ds(eb, CH)], buf, sem)

    def g_start(buf, idx_ref, sem):
        return pltpu.async_copy(x_hbm.at[idx_ref], buf, sem, add=True)

    def relu(buf):
        def rowbody(r, cr):
            for k in range(D // 16):
                v = buf[r, pl.ds(k * 16, 16)]
                buf[r, pl.ds(k * 16, 16)] = jnp.maximum(v, 0.0)
            return cr

        lax.fori_loop(0, CH, rowbody, 0)

    fill(sidx_a, sidx_all, 0)
    e_load(rows_a, sem_ea, 0).wait()
    ga0 = g_start(rows_a, sidx_a, sem_ga)
    eb0 = e_load(rows_b, sem_eb, 1)
    ga0.wait()
    eb0.wait()
    plsc.subcore_barrier()

    def pipe(i, carry):
        j = 2 * i
        fill(sidx_b, sidx_all, j + 1)
        gb = g_start(rows_b, sidx_b, sem_gb)
        fill(didx_a, didx_all, j)
        relu(rows_a)
        sa = pltpu.async_copy(rows_a, aggr_sh.at[didx_a], sem_sa, add=True)
        gb.wait()
        fill(didx_b, didx_all, j + 1)
        relu(rows_b)
        sa.wait()
        ea = e_load(rows_a, sem_ea, j + 2)
        sb = pltpu.async_copy(rows_b, aggr_sh.at[didx_b], sem_sb, add=True)
        ea.wait()
        fill(sidx_a, sidx_all, j + 2)
        ga = g_start(rows_a, sidx_a, sem_ga)
        sb.wait()
        eb = e_load(rows_b, sem_eb, j + 3)
        ga.wait()
        eb.wait()
        return carry

    lax.fori_loop(0, (NCHUNK - 1) // 2, pipe, 0)
    fill(didx_a, didx_all, NCHUNK - 1)
    relu(rows_a)
    pltpu.sync_copy(rows_a, aggr_sh.at[didx_a], add=True)
    plsc.subcore_barrier()

    for t in range(ROWS_PS // CH):
        r0 = row0 + t * CH
        pltpu.sync_copy(aggr_sh.at[pl.ds(r0, CH)], rows_a)
        pltpu.sync_copy(rows_a, out_hbm.at[c, pl.ds(r0, CH)])


_sc_msg = pl.kernel(
    _sc_msg_body,
    out_type=jax.ShapeDtypeStruct((NC, N_PAD, D), jnp.float32),
    mesh=plsc.VectorSubcoreMesh(core_axis_name="c", subcore_axis_name="s"),
    scratch_types=[
        pltpu.VMEM((EPT,), jnp.int32),
        pltpu.VMEM((EPT,), jnp.int32),
        pltpu.VMEM((CH,), jnp.int32),
        pltpu.VMEM((CH,), jnp.int32),
        pltpu.VMEM((CH,), jnp.int32),
        pltpu.VMEM((CH,), jnp.int32),
        pltpu.VMEM((CH, D), jnp.float32),
        pltpu.VMEM((CH, D), jnp.float32),
        pltpu.VMEM_SHARED((N_PAD, D), jnp.float32),
        pltpu.SemaphoreType.DMA,
        pltpu.SemaphoreType.DMA,
        pltpu.SemaphoreType.DMA,
        pltpu.SemaphoreType.DMA,
        pltpu.SemaphoreType.DMA,
        pltpu.SemaphoreType.DMA,
        pltpu.SemaphoreType.DMA,
    ],
)


def _node_body(residual, x_ref, p_ref, W1_ref, b1_ref, W2_ref, b2_ref,
               gamma_ref, beta_ref, out_ref):
    x = x_ref[...]
    hin = x + p_ref[0] + p_ref[1]
    t = jnp.maximum(jnp.dot(hin, W1_ref[...], preferred_element_type=jnp.float32) + b1_ref[...], 0.0)
    h2 = jnp.dot(t, W2_ref[...], preferred_element_type=jnp.float32) + b2_ref[...]
    scale = gamma_ref[...] * (1.0 / (1.0 + BN_EPS) ** 0.5)
    h2 = jnp.maximum(h2 * scale + beta_ref[...], 0.0)
    if residual:
        h2 = x + h2
    out_ref[...] = h2


def _node_update(x, parts, W1, b1, W2, b2, gamma, beta, residual):
    return pl.pallas_call(
        functools.partial(_node_body, residual),
        grid=(NB,),
        in_specs=[
            pl.BlockSpec((RB, D), lambda i: (i, 0)),
            pl.BlockSpec((NC, RB, D), lambda i: (0, i, 0)),
            pl.BlockSpec((D, D), lambda i: (0, 0)),
            pl.BlockSpec((1, D), lambda i: (0, 0)),
            pl.BlockSpec((D, D), lambda i: (0, 0)),
            pl.BlockSpec((1, D), lambda i: (0, 0)),
            pl.BlockSpec((1, D), lambda i: (0, 0)),
            pl.BlockSpec((1, D), lambda i: (0, 0)),
        ],
        out_specs=pl.BlockSpec((RB, D), lambda i: (i, 0)),
        out_shape=jax.ShapeDtypeStruct((N_PAD, D), jnp.float32),
    )(x, parts, W1, b1.reshape(1, D), W2, b2.reshape(1, D),
      gamma.reshape(1, D), beta.reshape(1, D))


def _node_pool_body(x_ref, p_ref, W1_ref, b1_ref, W2_ref, b2_ref,
                    gamma_ref, beta_ref, b_ref, Wm1_ref, bm1_ref, Wm2_ref,
                    bm2_ref, out_ref, sums, counts):
    i = pl.program_id(0)

    @pl.when(i == 0)
    def _():
        sums[...] = jnp.zeros_like(sums)
        counts[...] = jnp.zeros_like(counts)

    x = x_ref[...]
    hin = x + p_ref[0] + p_ref[1]
    t = jnp.maximum(jnp.dot(hin, W1_ref[...], preferred_element_type=jnp.float32) + b1_ref[...], 0.0)
    h2 = jnp.dot(t, W2_ref[...], preferred_element_type=jnp.float32) + b2_ref[...]
    scale = gamma_ref[...] * (1.0 / (1.0 + BN_EPS) ** 0.5)
    h1 = x + jnp.maximum(h2 * scale + beta_ref[...], 0.0)

    bvec = b_ref[0]
    oh = (jnp.broadcast_to(bvec, (G_SEG, RB))
          == lax.broadcasted_iota(jnp.int32, (G_SEG, RB), 0)).astype(jnp.float32)
    sums[...] += jnp.dot(oh, h1, preferred_element_type=jnp.float32)
    counts[...] += jnp.dot(oh, jnp.ones((RB, D), jnp.float32),
                           preferred_element_type=jnp.float32)

    @pl.when(i == pl.num_programs(0) - 1)
    def _():
        pooled = sums[...] / jnp.maximum(counts[...], 1.0)
        a = jnp.maximum(jnp.dot(pooled, Wm1_ref[...], preferred_element_type=jnp.float32)
                        + bm1_ref[...], 0.0)
        out_ref[...] = jnp.dot(a, Wm2_ref[...], preferred_element_type=jnp.float32) + bm2_ref[...]


def _node_pool_classify(x, parts, W1, b1, W2, b2, gamma, beta, batch3d,
                        Wm1, bm1, Wm2, bm2):
    nh = Wm1.shape[1]
    nout = Wm2.shape[1]
    return pl.pallas_call(
        _node_pool_body,
        grid=(NB,),
        in_specs=[
            pl.BlockSpec((RB, D), lambda i: (i, 0)),
            pl.BlockSpec((NC, RB, D), lambda i: (0, i, 0)),
            pl.BlockSpec((D, D), lambda i: (0, 0)),
            pl.BlockSpec((1, D), lambda i: (0, 0)),
            pl.BlockSpec((D, D), lambda i: (0, 0)),
            pl.BlockSpec((1, D), lambda i: (0, 0)),
            pl.BlockSpec((1, D), lambda i: (0, 0)),
            pl.BlockSpec((1, D), lambda i: (0, 0)),
            pl.BlockSpec((1, 1, RB), lambda i: (i, 0, 0)),
            pl.BlockSpec((D, nh), lambda i: (0, 0)),
            pl.BlockSpec((1, nh), lambda i: (0, 0)),
            pl.BlockSpec((nh, nout), lambda i: (0, 0)),
            pl.BlockSpec((1, nout), lambda i: (0, 0)),
        ],
        out_specs=pl.BlockSpec((G_SEG, nout), lambda i: (0, 0)),
        out_shape=jax.ShapeDtypeStruct((G_SEG, nout), jnp.float32),
        scratch_shapes=[
            pltpu.VMEM((G_SEG, D), jnp.float32),
            pltpu.VMEM((G_SEG, D), jnp.float32),
        ],
    )(x, parts, W1, b1.reshape(1, D), W2, b2.reshape(1, D),
      gamma.reshape(1, D), beta.reshape(1, D), batch3d,
      Wm1, bm1.reshape(1, nh), Wm2, bm2.reshape(1, nout))


def kernel(x, edge_index, edge_attr, batch, We0, be0, W1_0, b1_0, W2_0, b2_0,
           gamma0, beta0, We1, be1, W1_1, b1_1, W2_1, b2_1, gamma1, beta1,
           Wm1, bm1, Wm2, bm2):
    x_p = jnp.pad(x, ((0, N_PAD - N_NODES), (0, 0)))
    batch_p = jnp.pad(batch.astype(jnp.int32), (0, N_PAD - N_NODES),
                      constant_values=G_SEG).reshape(NB, 1, RB)
    src = edge_index[0].astype(jnp.int32).reshape(NW, EPT)
    dst = edge_index[1].astype(jnp.int32).reshape(NW, EPT)

    e0 = _edge_transform(edge_attr, We0, be0)
    e1 = _edge_transform(edge_attr, We1, be1)

    parts0 = _sc_msg(x_p, src, dst, e0)
    h0 = _node_update(x_p, parts0, W1_0, b1_0, W2_0, b2_0, gamma0, beta0,
                      residual=False)
    parts1 = _sc_msg(h0, src, dst, e1)
    return _node_pool_classify(h0, parts1, W1_1, b1_1, W2_1, b2_1,
                               gamma1, beta1, batch_p, Wm1, bm1, Wm2, bm2)


# speedup vs baseline: 1.0981x; 1.0981x over previous
import functools

import jax
import jax.numpy as jnp
from jax import lax
from jax.experimental import pallas as pl
from jax.experimental.pallas import tpu as pltpu
from jax.experimental.pallas import tpu_sc as plsc

N_NODES = 10000
E_EDGES = 320000
D = 128
DE = 16
G_SEG = 128
BN_EPS = 1e-05

NC, NS = 2, 16
NW = NC * NS
N_PAD = 10240
ROWS_PS = N_PAD // NS
EPT = E_EDGES // NW
CH = 80
NCHUNK = EPT // CH

NB = 8
RB = N_PAD // NB
EB = 3200


def _edge_body(ea_ref, W_ref, b_ref, e_ref):
    a = ea_ref[...]
    e_ref[...] = jnp.dot(a, W_ref[...], preferred_element_type=jnp.float32) + b_ref[...]


def _edge_transform(edge_attr, We, be):
    grid = E_EDGES // EB
    return pl.pallas_call(
        _edge_body,
        grid=(grid,),
        in_specs=[
            pl.BlockSpec((EB, DE), lambda i: (i, 0)),
            pl.BlockSpec((DE, D), lambda i: (0, 0)),
            pl.BlockSpec((1, D), lambda i: (0, 0)),
        ],
        out_specs=pl.BlockSpec((EB, D), lambda i: (i, 0)),
        out_shape=jax.ShapeDtypeStruct((E_EDGES, D), jnp.float32),
    )(edge_attr, We, be.reshape(1, D))


def _sc_msg_body(x_hbm, src_hbm, dst_hbm, e_hbm, out_hbm,
                 sidx_all, didx_all, sidx_a, didx_a, sidx_b, didx_b,
                 rows_a, rows_b, aggr_sh,
                 sem_l, sem_ea, sem_eb, sem_ga, sem_gb, sem_sa, sem_sb):
    c = lax.axis_index("c")
    s = lax.axis_index("s")
    w = c * NS + s
    row0 = s * ROWS_PS
    ebase0 = w * EPT

    pltpu.sync_copy(src_hbm.at[w], sidx_all)
    pltpu.sync_copy(dst_hbm.at[w], didx_all)

    def zstore(r, carry):
        for k in range(D // 16):
            rows_a[r, pl.ds(k * 16, 16)] = jnp.zeros((16,), jnp.float32)
        return carry

    lax.fori_loop(0, CH, zstore, 0)
    for t in range(ROWS_PS // CH):
        pltpu.sync_copy(rows_a, aggr_sh.at[pl.ds(row0 + t * CH, CH)])

    def fill(dst_ref, src_ref, j):
        for k in range(CH // 16):
            dst_ref[pl.ds(k * 16, 16)] = src_ref[pl.ds(j * CH + k * 16, 16)]

    def e_load(buf, sem, j):
        eb = ebase0 + lax.rem(j, NCHUNK) * CH
        return pltpu.async_copy(e_hbm.at[pl.ds(eb, CH)], buf, sem)

    def g_start(buf, idx_ref, sem):
        return pltpu.async_copy(x_hbm.at[idx_ref], buf, sem, add=True)

    def relu(buf):
        @plsc.parallel_loop(0, CH, 1, unroll=2)
        def _rowbody(r):
            for k in range(D // 16):
                v = buf[r, pl.ds(k * 16, 16)]
                buf[r, pl.ds(k * 16, 16)] = jnp.maximum(v, 0.0)

    fill(sidx_a, sidx_all, 0)
    e_load(rows_a, sem_ea, 0).wait()
    ga0 = g_start(rows_a, sidx_a, sem_ga)
    eb0 = e_load(rows_b, sem_eb, 1)
    ga0.wait()
    eb0.wait()
    plsc.subcore_barrier()

    def pipe(i, carry):
        j = 2 * i
        fill(sidx_b, sidx_all, j + 1)
        gb = g_start(rows_b, sidx_b, sem_gb)
        fill(didx_a, didx_all, j)
        relu(rows_a)
        pltpu.sync_copy(rows_a, aggr_sh.at[didx_a], add=True)
        ea = e_load(rows_a, sem_ea, j + 2)
        gb.wait()
        fill(didx_b, didx_all, j + 1)
        relu(rows_b)
        pltpu.sync_copy(rows_b, aggr_sh.at[didx_b], add=True)
        ea.wait()
        fill(sidx_a, sidx_all, j + 2)
        ga = g_start(rows_a, sidx_a, sem_ga)
        eb = e_load(rows_b, sem_eb, j + 3)
        ga.wait()
        eb.wait()
        return carry

    lax.fori_loop(0, (NCHUNK - 1) // 2, pipe, 0)
    fill(didx_a, didx_all, NCHUNK - 1)
    relu(rows_a)
    pltpu.sync_copy(rows_a, aggr_sh.at[didx_a], add=True)
    plsc.subcore_barrier()

    for t in range(ROWS_PS // CH):
        r0 = row0 + t * CH
        pltpu.sync_copy(aggr_sh.at[pl.ds(r0, CH)], rows_a)
        pltpu.sync_copy(rows_a, out_hbm.at[c, pl.ds(r0, CH)])


_sc_msg = pl.kernel(
    _sc_msg_body,
    out_type=jax.ShapeDtypeStruct((NC, N_PAD, D), jnp.float32),
    mesh=plsc.VectorSubcoreMesh(core_axis_name="c", subcore_axis_name="s"),
    scratch_types=[
        pltpu.VMEM((EPT,), jnp.int32),
        pltpu.VMEM((EPT,), jnp.int32),
        pltpu.VMEM((CH,), jnp.int32),
        pltpu.VMEM((CH,), jnp.int32),
        pltpu.VMEM((CH,), jnp.int32),
        pltpu.VMEM((CH,), jnp.int32),
        pltpu.VMEM((CH, D), jnp.float32),
        pltpu.VMEM((CH, D), jnp.float32),
        pltpu.VMEM_SHARED((N_PAD, D), jnp.float32),
        pltpu.SemaphoreType.DMA,
        pltpu.SemaphoreType.DMA,
        pltpu.SemaphoreType.DMA,
        pltpu.SemaphoreType.DMA,
        pltpu.SemaphoreType.DMA,
        pltpu.SemaphoreType.DMA,
        pltpu.SemaphoreType.DMA,
    ],
)


def _node_body(residual, x_ref, p_ref, W1_ref, b1_ref, W2_ref, b2_ref,
               gamma_ref, beta_ref, out_ref):
    x = x_ref[...]
    hin = x + p_ref[0] + p_ref[1]
    t = jnp.maximum(jnp.dot(hin, W1_ref[...], preferred_element_type=jnp.float32) + b1_ref[...], 0.0)
    h2 = jnp.dot(t, W2_ref[...], preferred_element_type=jnp.float32) + b2_ref[...]
    scale = gamma_ref[...] * (1.0 / (1.0 + BN_EPS) ** 0.5)
    h2 = jnp.maximum(h2 * scale + beta_ref[...], 0.0)
    if residual:
        h2 = x + h2
    out_ref[...] = h2


def _node_update(x, parts, W1, b1, W2, b2, gamma, beta, residual):
    return pl.pallas_call(
        functools.partial(_node_body, residual),
        grid=(NB,),
        in_specs=[
            pl.BlockSpec((RB, D), lambda i: (i, 0)),
            pl.BlockSpec((NC, RB, D), lambda i: (0, i, 0)),
            pl.BlockSpec((D, D), lambda i: (0, 0)),
            pl.BlockSpec((1, D), lambda i: (0, 0)),
            pl.BlockSpec((D, D), lambda i: (0, 0)),
            pl.BlockSpec((1, D), lambda i: (0, 0)),
            pl.BlockSpec((1, D), lambda i: (0, 0)),
            pl.BlockSpec((1, D), lambda i: (0, 0)),
        ],
        out_specs=pl.BlockSpec((RB, D), lambda i: (i, 0)),
        out_shape=jax.ShapeDtypeStruct((N_PAD, D), jnp.float32),
    )(x, parts, W1, b1.reshape(1, D), W2, b2.reshape(1, D),
      gamma.reshape(1, D), beta.reshape(1, D))


def _node_pool_body(x_ref, p_ref, W1_ref, b1_ref, W2_ref, b2_ref,
                    gamma_ref, beta_ref, b_ref, Wm1_ref, bm1_ref, Wm2_ref,
                    bm2_ref, out_ref, sums, counts):
    i = pl.program_id(0)

    @pl.when(i == 0)
    def _():
        sums[...] = jnp.zeros_like(sums)
        counts[...] = jnp.zeros_like(counts)

    x = x_ref[...]
    hin = x + p_ref[0] + p_ref[1]
    t = jnp.maximum(jnp.dot(hin, W1_ref[...], preferred_element_type=jnp.float32) + b1_ref[...], 0.0)
    h2 = jnp.dot(t, W2_ref[...], preferred_element_type=jnp.float32) + b2_ref[...]
    scale = gamma_ref[...] * (1.0 / (1.0 + BN_EPS) ** 0.5)
    h1 = x + jnp.maximum(h2 * scale + beta_ref[...], 0.0)

    bvec = b_ref[0]
    oh = (jnp.broadcast_to(bvec, (G_SEG, RB))
          == lax.broadcasted_iota(jnp.int32, (G_SEG, RB), 0)).astype(jnp.float32)
    sums[...] += jnp.dot(oh, h1, preferred_element_type=jnp.float32)
    counts[...] += jnp.dot(oh, jnp.ones((RB, D), jnp.float32),
                           preferred_element_type=jnp.float32)

    @pl.when(i == pl.num_programs(0) - 1)
    def _():
        pooled = sums[...] / jnp.maximum(counts[...], 1.0)
        a = jnp.maximum(jnp.dot(pooled, Wm1_ref[...], preferred_element_type=jnp.float32)
                        + bm1_ref[...], 0.0)
        out_ref[...] = jnp.dot(a, Wm2_ref[...], preferred_element_type=jnp.float32) + bm2_ref[...]


def _node_pool_classify(x, parts, W1, b1, W2, b2, gamma, beta, batch3d,
                        Wm1, bm1, Wm2, bm2):
    nh = Wm1.shape[1]
    nout = Wm2.shape[1]
    return pl.pallas_call(
        _node_pool_body,
        grid=(NB,),
        in_specs=[
            pl.BlockSpec((RB, D), lambda i: (i, 0)),
            pl.BlockSpec((NC, RB, D), lambda i: (0, i, 0)),
            pl.BlockSpec((D, D), lambda i: (0, 0)),
            pl.BlockSpec((1, D), lambda i: (0, 0)),
            pl.BlockSpec((D, D), lambda i: (0, 0)),
            pl.BlockSpec((1, D), lambda i: (0, 0)),
            pl.BlockSpec((1, D), lambda i: (0, 0)),
            pl.BlockSpec((1, D), lambda i: (0, 0)),
            pl.BlockSpec((1, 1, RB), lambda i: (i, 0, 0)),
            pl.BlockSpec((D, nh), lambda i: (0, 0)),
            pl.BlockSpec((1, nh), lambda i: (0, 0)),
            pl.BlockSpec((nh, nout), lambda i: (0, 0)),
            pl.BlockSpec((1, nout), lambda i: (0, 0)),
        ],
        out_specs=pl.BlockSpec((G_SEG, nout), lambda i: (0, 0)),
        out_shape=jax.ShapeDtypeStruct((G_SEG, nout), jnp.float32),
        scratch_shapes=[
            pltpu.VMEM((G_SEG, D), jnp.float32),
            pltpu.VMEM((G_SEG, D), jnp.float32),
        ],
    )(x, parts, W1, b1.reshape(1, D), W2, b2.reshape(1, D),
      gamma.reshape(1, D), beta.reshape(1, D), batch3d,
      Wm1, bm1.reshape(1, nh), Wm2, bm2.reshape(1, nout))


def kernel(x, edge_index, edge_attr, batch, We0, be0, W1_0, b1_0, W2_0, b2_0,
           gamma0, beta0, We1, be1, W1_1, b1_1, W2_1, b2_1, gamma1, beta1,
           Wm1, bm1, Wm2, bm2):
    x_p = jnp.pad(x, ((0, N_PAD - N_NODES), (0, 0)))
    batch_p = jnp.pad(batch.astype(jnp.int32), (0, N_PAD - N_NODES),
                      constant_values=G_SEG).reshape(NB, 1, RB)
    src = edge_index[0].astype(jnp.int32).reshape(NW, EPT)
    dst = edge_index[1].astype(jnp.int32).reshape(NW, EPT)

    e0 = _edge_transform(edge_attr, We0, be0)
    e1 = _edge_transform(edge_attr, We1, be1)

    parts0 = _sc_msg(x_p, src, dst, e0)
    h0 = _node_update(x_p, parts0, W1_0, b1_0, W2_0, b2_0, gamma0, beta0,
                      residual=False)
    parts1 = _sc_msg(h0, src, dst, e1)
    return _node_pool_classify(h0, parts1, W1_1, b1_1, W2_1, b2_1,
                               gamma1, beta1, batch_p, Wm1, bm1, Wm2, bm2)


# speedup vs baseline: 1.1655x; 1.0614x over previous
import functools

import jax
import jax.numpy as jnp
from jax import lax
from jax.experimental import pallas as pl
from jax.experimental.pallas import tpu as pltpu
from jax.experimental.pallas import tpu_sc as plsc

N_NODES = 10000
E_EDGES = 320000
D = 128
DE = 16
G_SEG = 128
BN_EPS = 1e-05

NC, NS = 2, 16
NW = NC * NS
N_PAD = 10240
ROWS_PS = N_PAD // NS
EPT = E_EDGES // NW
CH = 128
NFULL = EPT // CH
TAIL = EPT - NFULL * CH

NB = 8
RB = N_PAD // NB
EB = 3200


def _edge_body(ea_ref, W_ref, b_ref, e_ref):
    a = ea_ref[...]
    e_ref[...] = jnp.dot(a, W_ref[...], preferred_element_type=jnp.float32) + b_ref[...]


def _edge_transform(edge_attr, We, be):
    grid = E_EDGES // EB
    return pl.pallas_call(
        _edge_body,
        grid=(grid,),
        in_specs=[
            pl.BlockSpec((EB, DE), lambda i: (i, 0)),
            pl.BlockSpec((DE, D), lambda i: (0, 0)),
            pl.BlockSpec((1, D), lambda i: (0, 0)),
        ],
        out_specs=pl.BlockSpec((EB, D), lambda i: (i, 0)),
        out_shape=jax.ShapeDtypeStruct((E_EDGES, D), jnp.float32),
    )(edge_attr, We, be.reshape(1, D))


def _sc_msg_body(x_hbm, src_hbm, dst_hbm, e_hbm, out_hbm,
                 sidx_a, didx_a, sidx_b, didx_b, sidx_t, didx_t,
                 rows_a, rows_b, aggr_sh,
                 sem_l, sem_ea, sem_eb, sem_ga, sem_gb, sem_ia, sem_ib):
    c = lax.axis_index("c")
    s = lax.axis_index("s")
    w = c * NS + s
    row0 = s * ROWS_PS
    ebase0 = w * EPT

    def zstore(r, carry):
        for k in range(D // 16):
            rows_a[r, pl.ds(k * 16, 16)] = jnp.zeros((16,), jnp.float32)
        return carry

    lax.fori_loop(0, CH, zstore, 0)
    for t in range(ROWS_PS // CH):
        pltpu.sync_copy(rows_a, aggr_sh.at[pl.ds(row0 + t * CH, CH)])

    def i_load(sbuf, dbuf, sem, j, n=CH):
        eb = ebase0 + j * CH
        c1 = pltpu.async_copy(src_hbm.at[pl.ds(eb, n)], sbuf, sem)
        c2 = pltpu.async_copy(dst_hbm.at[pl.ds(eb, n)], dbuf, sem)
        return c1, c2

    def e_load(buf, sem, j):
        eb = ebase0 + j * CH
        return pltpu.async_copy(e_hbm.at[pl.ds(eb, CH)], buf, sem)

    def g_start(buf, idx_ref, sem):
        return pltpu.async_copy(x_hbm.at[idx_ref], buf, sem, add=True)

    def relu(buf):
        @plsc.parallel_loop(0, CH, 1, unroll=2)
        def _rowbody(r):
            for k in range(D // 16):
                v = buf[r, pl.ds(k * 16, 16)]
                buf[r, pl.ds(k * 16, 16)] = jnp.maximum(v, 0.0)

    for cp in i_load(sidx_a, didx_a, sem_ia, 0):
        cp.wait()
    e_load(rows_a, sem_ea, 0).wait()
    ga0 = g_start(rows_a, sidx_a, sem_ga)
    ib1, ib2 = i_load(sidx_b, didx_b, sem_ib, 1)
    eb0 = e_load(rows_b, sem_eb, 1)
    ga0.wait()
    eb0.wait()
    ib1.wait()
    ib2.wait()
    plsc.subcore_barrier()

    def pipe(i, carry):
        j = 2 * i
        gb = g_start(rows_b, sidx_b, sem_gb)
        relu(rows_a)
        pltpu.sync_copy(rows_a, aggr_sh.at[didx_a], add=True)
        ia1, ia2 = i_load(sidx_a, didx_a, sem_ia, j + 2)
        ea = e_load(rows_a, sem_ea, j + 2)
        gb.wait()
        relu(rows_b)
        pltpu.sync_copy(rows_b, aggr_sh.at[didx_b], add=True)
        ea.wait()
        ia1.wait()
        ia2.wait()
        ga = g_start(rows_a, sidx_a, sem_ga)
        ib1_, ib2_ = i_load(sidx_b, didx_b, sem_ib, j + 3)
        eb = e_load(rows_b, sem_eb, j + 3)
        ga.wait()
        eb.wait()
        ib1_.wait()
        ib2_.wait()
        return carry

    lax.fori_loop(0, NFULL // 2 - 1, pipe, 0)
    gb = g_start(rows_b, sidx_b, sem_gb)
    relu(rows_a)
    pltpu.sync_copy(rows_a, aggr_sh.at[didx_a], add=True)
    gb.wait()
    relu(rows_b)
    pltpu.sync_copy(rows_b, aggr_sh.at[didx_b], add=True)
    for cp in i_load(sidx_t, didx_t, sem_ia, NFULL, n=TAIL):
        cp.wait()
    pltpu.sync_copy(e_hbm.at[pl.ds(ebase0 + NFULL * CH, TAIL)],
                    rows_a.at[pl.ds(0, TAIL)])
    pltpu.async_copy(x_hbm.at[sidx_t], rows_a.at[pl.ds(0, TAIL)],
                     sem_ga, add=True).wait()
    for r in range(TAIL):
        for k in range(D // 16):
            v = rows_a[r, pl.ds(k * 16, 16)]
            rows_a[r, pl.ds(k * 16, 16)] = jnp.maximum(v, 0.0)
    pltpu.sync_copy(rows_a.at[pl.ds(0, TAIL)], aggr_sh.at[didx_t], add=True)
    plsc.subcore_barrier()

    for t in range(ROWS_PS // CH):
        r0 = row0 + t * CH
        pltpu.sync_copy(aggr_sh.at[pl.ds(r0, CH)], rows_a)
        pltpu.sync_copy(rows_a, out_hbm.at[c, pl.ds(r0, CH)])


_sc_msg = pl.kernel(
    _sc_msg_body,
    out_type=jax.ShapeDtypeStruct((NC, N_PAD, D), jnp.float32),
    mesh=plsc.VectorSubcoreMesh(core_axis_name="c", subcore_axis_name="s"),
    scratch_types=[
        pltpu.VMEM((CH,), jnp.int32),
        pltpu.VMEM((CH,), jnp.int32),
        pltpu.VMEM((CH,), jnp.int32),
        pltpu.VMEM((CH,), jnp.int32),
        pltpu.VMEM((TAIL,), jnp.int32),
        pltpu.VMEM((TAIL,), jnp.int32),
        pltpu.VMEM((CH, D), jnp.float32),
        pltpu.VMEM((CH, D), jnp.float32),
        pltpu.VMEM_SHARED((N_PAD, D), jnp.float32),
        pltpu.SemaphoreType.DMA,
        pltpu.SemaphoreType.DMA,
        pltpu.SemaphoreType.DMA,
        pltpu.SemaphoreType.DMA,
        pltpu.SemaphoreType.DMA,
        pltpu.SemaphoreType.DMA,
        pltpu.SemaphoreType.DMA,
    ],
)


def _node_body(residual, x_ref, p_ref, W1_ref, b1_ref, W2_ref, b2_ref,
               gamma_ref, beta_ref, out_ref):
    x = x_ref[...]
    hin = x + p_ref[0] + p_ref[1]
    t = jnp.maximum(jnp.dot(hin, W1_ref[...], preferred_element_type=jnp.float32) + b1_ref[...], 0.0)
    h2 = jnp.dot(t, W2_ref[...], preferred_element_type=jnp.float32) + b2_ref[...]
    scale = gamma_ref[...] * (1.0 / (1.0 + BN_EPS) ** 0.5)
    h2 = jnp.maximum(h2 * scale + beta_ref[...], 0.0)
    if residual:
        h2 = x + h2
    out_ref[...] = h2


def _node_update(x, parts, W1, b1, W2, b2, gamma, beta, residual):
    return pl.pallas_call(
        functools.partial(_node_body, residual),
        grid=(NB,),
        in_specs=[
            pl.BlockSpec((RB, D), lambda i: (i, 0)),
            pl.BlockSpec((NC, RB, D), lambda i: (0, i, 0)),
            pl.BlockSpec((D, D), lambda i: (0, 0)),
            pl.BlockSpec((1, D), lambda i: (0, 0)),
            pl.BlockSpec((D, D), lambda i: (0, 0)),
            pl.BlockSpec((1, D), lambda i: (0, 0)),
            pl.BlockSpec((1, D), lambda i: (0, 0)),
            pl.BlockSpec((1, D), lambda i: (0, 0)),
        ],
        out_specs=pl.BlockSpec((RB, D), lambda i: (i, 0)),
        out_shape=jax.ShapeDtypeStruct((N_PAD, D), jnp.float32),
    )(x, parts, W1, b1.reshape(1, D), W2, b2.reshape(1, D),
      gamma.reshape(1, D), beta.reshape(1, D))


def _node_pool_body(x_ref, p_ref, W1_ref, b1_ref, W2_ref, b2_ref,
                    gamma_ref, beta_ref, b_ref, Wm1_ref, bm1_ref, Wm2_ref,
                    bm2_ref, out_ref, sums, counts):
    i = pl.program_id(0)

    @pl.when(i == 0)
    def _():
        sums[...] = jnp.zeros_like(sums)
        counts[...] = jnp.zeros_like(counts)

    x = x_ref[...]
    hin = x + p_ref[0] + p_ref[1]
    t = jnp.maximum(jnp.dot(hin, W1_ref[...], preferred_element_type=jnp.float32) + b1_ref[...], 0.0)
    h2 = jnp.dot(t, W2_ref[...], preferred_element_type=jnp.float32) + b2_ref[...]
    scale = gamma_ref[...] * (1.0 / (1.0 + BN_EPS) ** 0.5)
    h1 = x + jnp.maximum(h2 * scale + beta_ref[...], 0.0)

    bvec = b_ref[0]
    oh = (jnp.broadcast_to(bvec, (G_SEG, RB))
          == lax.broadcasted_iota(jnp.int32, (G_SEG, RB), 0)).astype(jnp.float32)
    sums[...] += jnp.dot(oh, h1, preferred_element_type=jnp.float32)
    counts[...] += jnp.dot(oh, jnp.ones((RB, D), jnp.float32),
                           preferred_element_type=jnp.float32)

    @pl.when(i == pl.num_programs(0) - 1)
    def _():
        pooled = sums[...] / jnp.maximum(counts[...], 1.0)
        a = jnp.maximum(jnp.dot(pooled, Wm1_ref[...], preferred_element_type=jnp.float32)
                        + bm1_ref[...], 0.0)
        out_ref[...] = jnp.dot(a, Wm2_ref[...], preferred_element_type=jnp.float32) + bm2_ref[...]


def _node_pool_classify(x, parts, W1, b1, W2, b2, gamma, beta, batch3d,
                        Wm1, bm1, Wm2, bm2):
    nh = Wm1.shape[1]
    nout = Wm2.shape[1]
    return pl.pallas_call(
        _node_pool_body,
        grid=(NB,),
        in_specs=[
            pl.BlockSpec((RB, D), lambda i: (i, 0)),
            pl.BlockSpec((NC, RB, D), lambda i: (0, i, 0)),
            pl.BlockSpec((D, D), lambda i: (0, 0)),
            pl.BlockSpec((1, D), lambda i: (0, 0)),
            pl.BlockSpec((D, D), lambda i: (0, 0)),
            pl.BlockSpec((1, D), lambda i: (0, 0)),
            pl.BlockSpec((1, D), lambda i: (0, 0)),
            pl.BlockSpec((1, D), lambda i: (0, 0)),
            pl.BlockSpec((1, 1, RB), lambda i: (i, 0, 0)),
            pl.BlockSpec((D, nh), lambda i: (0, 0)),
            pl.BlockSpec((1, nh), lambda i: (0, 0)),
            pl.BlockSpec((nh, nout), lambda i: (0, 0)),
            pl.BlockSpec((1, nout), lambda i: (0, 0)),
        ],
        out_specs=pl.BlockSpec((G_SEG, nout), lambda i: (0, 0)),
        out_shape=jax.ShapeDtypeStruct((G_SEG, nout), jnp.float32),
        scratch_shapes=[
            pltpu.VMEM((G_SEG, D), jnp.float32),
            pltpu.VMEM((G_SEG, D), jnp.float32),
        ],
    )(x, parts, W1, b1.reshape(1, D), W2, b2.reshape(1, D),
      gamma.reshape(1, D), beta.reshape(1, D), batch3d,
      Wm1, bm1.reshape(1, nh), Wm2, bm2.reshape(1, nout))


def kernel(x, edge_index, edge_attr, batch, We0, be0, W1_0, b1_0, W2_0, b2_0,
           gamma0, beta0, We1, be1, W1_1, b1_1, W2_1, b2_1, gamma1, beta1,
           Wm1, bm1, Wm2, bm2):
    x_p = jnp.pad(x, ((0, N_PAD - N_NODES), (0, 0)))
    batch_p = jnp.pad(batch.astype(jnp.int32), (0, N_PAD - N_NODES),
                      constant_values=G_SEG).reshape(NB, 1, RB)
    src = edge_index[0].astype(jnp.int32)
    dst = edge_index[1].astype(jnp.int32)

    e0 = _edge_transform(edge_attr, We0, be0)
    e1 = _edge_transform(edge_attr, We1, be1)

    parts0 = _sc_msg(x_p, src, dst, e0)
    h0 = _node_update(x_p, parts0, W1_0, b1_0, W2_0, b2_0, gamma0, beta0,
                      residual=False)
    parts1 = _sc_msg(h0, src, dst, e1)
    return _node_pool_classify(h0, parts1, W1_1, b1_1, W2_1, b2_1,
                               gamma1, beta1, batch_p, Wm1, bm1, Wm2, bm2)
